# Initial kernel scaffold; baseline (speedup 1.0000x reference)
import jax
import jax.numpy as jnp
from jax.experimental import pallas as pl


def kernel(queries, keys, values):
    raise NotImplementedError("write your pallas kernel here")


# speedup vs baseline: 4.9880x; 4.9880x over previous
import math

import numpy as np
import jax
import jax.numpy as jnp
from jax.experimental import pallas as pl
from jax.experimental.pallas import tpu as pltpu

_L = 2048
_H = 12
_D = 64
_SAMPLE = 40
_TOPU = 40
_SCALE = 1.0 / math.sqrt(_D)

_RB = 256
_NRB = _L // _RB
_CB = 128
_SUB = _L // 8


def _count_matrix():
    idx = np.asarray(
        jax.random.randint(jax.random.key(42), (_L, _SAMPLE), 0, _L))
    c = np.zeros((_L, _L), np.float32)
    np.add.at(c, (np.arange(_L)[:, None], idx), 1.0)
    return c


_C = _count_matrix()


def _m_kernel(q_ref, k_ref, c_ref, m_ref):
    q = q_ref[0]
    k = k_ref[0]
    c = c_ref[...]
    s = jax.lax.dot_general(q, k, (((1,), (1,)), ((), ())),
                            preferred_element_type=jnp.float32)
    smax = jnp.max(jnp.where(c > 0.0, s, -jnp.inf), axis=1, keepdims=True)
    ssum = jnp.sum(s * c, axis=1, keepdims=True)
    m_ref[0] = smax - ssum * (1.0 / _L)


def _attn_kernel(m_ref, q_ref, k_ref, v_ref, o_ref, mtop_ref):
    m2 = m_ref[0]
    gi = (jax.lax.broadcasted_iota(jnp.int32, (8, _SUB), 0) * _SUB
          + jax.lax.broadcasted_iota(jnp.int32, (8, _SUB), 1))

    def body(u, mm):
        mx = jnp.max(mm)
        idx = jnp.min(jnp.where(mm == mx, gi, _L))
        mtop_ref[pl.ds(u, 1), :] = idx.astype(jnp.float32)[None, None]
        return jnp.where(gi == idx, -jnp.inf, mm)

    jax.lax.fori_loop(0, _TOPU, body, m2)

    q = q_ref[0]
    k = k_ref[0]
    v = v_ref[0]
    mtop = mtop_ref[...]
    lane = jax.lax.broadcasted_iota(
        jnp.int32, (_TOPU, _L), 1).astype(jnp.float32)
    oh = (lane == mtop).astype(jnp.float32)

    q_red = jnp.dot(oh, q, preferred_element_type=jnp.float32)
    s = jax.lax.dot_general(q_red, k, (((1,), (1,)), ((), ())),
                            preferred_element_type=jnp.float32) * _SCALE
    s = jnp.where(lane > mtop, -jnp.inf, s)
    smx = jnp.max(s, axis=1, keepdims=True)
    p = jnp.exp(s - smx)
    attn = p / jnp.sum(p, axis=1, keepdims=True)
    upd = jnp.dot(attn, v, preferred_element_type=jnp.float32)

    tril = (jax.lax.broadcasted_iota(jnp.int32, (_CB, _CB), 0)
            >= jax.lax.broadcasted_iota(jnp.int32, (_CB, _CB), 1)
            ).astype(jnp.float32)
    blocks = []
    carry = jnp.zeros((1, _D), jnp.float32)
    for b in range(_L // _CB):
        blk = v[b * _CB:(b + 1) * _CB]
        blocks.append(
            jnp.dot(tril, blk, preferred_element_type=jnp.float32) + carry)
        carry = carry + jnp.sum(blk, axis=0, keepdims=True)
    ctx = jnp.concatenate(blocks, axis=0)

    scat = jax.lax.dot_general(oh, upd, (((0,), (0,)), ((), ())),
                               preferred_element_type=jnp.float32)
    selc = jax.lax.dot_general(oh, jnp.ones((_TOPU, _D), jnp.float32),
                               (((0,), (0,)), ((), ())),
                               preferred_element_type=jnp.float32)
    o_ref[0] = jnp.where(selc > 0.5, scat, ctx)


def kernel(queries, keys, values):
    assert queries.shape == (1, _L, _H, _D), queries.shape
    qh = jnp.transpose(queries[0], (1, 0, 2))
    kh = jnp.transpose(keys[0], (1, 0, 2))
    vh = jnp.transpose(values[0], (1, 0, 2))
    c = jnp.asarray(_C)

    m3 = pl.pallas_call(
        _m_kernel,
        grid=(_NRB, _H),
        in_specs=[
            pl.BlockSpec((1, _RB, _D), lambda rb, h: (h, rb, 0)),
            pl.BlockSpec((1, _L, _D), lambda rb, h: (h, 0, 0)),
            pl.BlockSpec((_RB, _L), lambda rb, h: (rb, 0)),
        ],
        out_specs=pl.BlockSpec((1, _RB, 1), lambda rb, h: (h * _NRB + rb, 0, 0)),
        out_shape=jax.ShapeDtypeStruct((_H * _NRB, _RB, 1), jnp.float32),
    )(qh, kh, c)

    m = m3.reshape(_H, _L).reshape(_H, 8, _SUB)

    ctx = pl.pallas_call(
        _attn_kernel,
        grid=(_H,),
        in_specs=[
            pl.BlockSpec((1, 8, _SUB), lambda h: (h, 0, 0)),
            pl.BlockSpec((1, _L, _D), lambda h: (h, 0, 0)),
            pl.BlockSpec((1, _L, _D), lambda h: (h, 0, 0)),
            pl.BlockSpec((1, _L, _D), lambda h: (h, 0, 0)),
        ],
        out_specs=pl.BlockSpec((1, _L, _D), lambda h: (h, 0, 0)),
        out_shape=jax.ShapeDtypeStruct((_H, _L, _D), jnp.float32),
        scratch_shapes=[pltpu.VMEM((_TOPU, 1), jnp.float32)],
    )(m, qh, kh, vh)

    return jnp.transpose(ctx, (1, 0, 2))[None]
